# baseline (speedup 1.0000x reference)
import functools

import jax
import jax.numpy as jnp
from jax import lax
from jax.experimental import pallas as pl
from jax.experimental.pallas import tpu as pltpu
from jax.experimental.pallas import tpu_sc as plsc

_BATCH = 16384
_NUM_EXPERTS = 64
_NUM_CLASSES = 16
_TOP_K = 8
_IN_DIM = _NUM_EXPERTS * _NUM_CLASSES
_BLOCK = 2048

_SC_INFO = plsc.get_sparse_core_info()
_NC = _SC_INFO.num_cores
_NS = _SC_INFO.num_subcores
_NW = _NC * _NS
_IDX_CHUNK = 128


def _gating_kernel(x_ref, w1_ref, b1_ref, w2_ref, b2_ref, w3_ref, b3_ref,
                   idx_ref, wts_ref):
    x = x_ref[...].reshape(_BLOCK, _IN_DIM)
    h = jnp.maximum(
        jnp.dot(x.astype(jnp.bfloat16), w1_ref[...].astype(jnp.bfloat16),
                preferred_element_type=jnp.float32)
        + b1_ref[...], 0.0)
    h = jnp.maximum(
        jnp.dot(h.astype(jnp.bfloat16), w2_ref[...].astype(jnp.bfloat16),
                preferred_element_type=jnp.float32)
        + b2_ref[...], 0.0)
    logits = (jnp.dot(h.astype(jnp.bfloat16), w3_ref[...].astype(jnp.bfloat16),
                      preferred_element_type=jnp.float32)
              + b3_ref[...])

    lane = lax.broadcasted_iota(jnp.int32, (_BLOCK, _NUM_EXPERTS), 1)
    ew = logits
    m = None
    idx_cols = []
    wts_cols = []
    for _ in range(_TOP_K):
        mx = jnp.max(ew, axis=1, keepdims=True)
        if m is None:
            m = mx
        hit = ew == mx
        idx_cols.append(jnp.max(jnp.where(hit, lane, -1), axis=1,
                                keepdims=True))
        wts_cols.append(jnp.exp(mx - m))
        ew = jnp.where(hit, -jnp.inf, ew)

    e_idx = jnp.concatenate(idx_cols, axis=1)
    wts = jnp.concatenate(wts_cols, axis=1)
    wts = wts / jnp.sum(wts, axis=1, keepdims=True)

    row = (pl.program_id(0) * _BLOCK
           + lax.broadcasted_iota(jnp.int32, (_BLOCK, _TOP_K), 0))
    idx_ref[...] = row * _NUM_EXPERTS + e_idx
    wts_ref[...] = wts


def _tc_gating(flat8, W1, b1, W2, b2, W3, b3):
    B = flat8.shape[0]
    full = lambda shape: pl.BlockSpec(shape, lambda i: (0,) * len(shape))
    return pl.pallas_call(
        _gating_kernel,
        grid=(B // _BLOCK,),
        in_specs=[
            pl.BlockSpec((_BLOCK, 8, 128), lambda i: (i, 0, 0)),
            full(W1.shape), full(b1.shape),
            full(W2.shape), full(b2.shape),
            full(W3.shape), full(b3.shape),
        ],
        out_specs=[
            pl.BlockSpec((_BLOCK, _TOP_K), lambda i: (i, 0)),
            pl.BlockSpec((_BLOCK, _TOP_K), lambda i: (i, 0)),
        ],
        out_shape=[
            jax.ShapeDtypeStruct((B, _TOP_K), jnp.int32),
            jax.ShapeDtypeStruct((B, _TOP_K), jnp.float32),
        ],
    )(flat8, W1, b1, W2, b2, W3, b3)


def _splat_lane(vec, lane):
    return lax.gather(
        vec,
        jnp.full((16, 1), lane, jnp.int32),
        lax.GatherDimensionNumbers(offset_dims=(),
                                   collapsed_slice_dims=(0,),
                                   start_index_map=(0,)),
        slice_sizes=(1,),
        mode=lax.GatherScatterMode.PROMISE_IN_BOUNDS,
    )


def _sc_combine(table, idx2d, wflat):
    B = wflat.shape[0] // _TOP_K
    rows_per_w = B // _NW
    g_per_w = rows_per_w * _TOP_K
    chunks = g_per_w // _IDX_CHUNK

    @functools.partial(
        pl.kernel,
        mesh=plsc.VectorSubcoreMesh(core_axis_name="c", subcore_axis_name="s"),
        compiler_params=pltpu.CompilerParams(use_tc_tiling_on_sc=False),
        out_type=jax.ShapeDtypeStruct((B, _NUM_CLASSES), jnp.float32),
        scratch_types=[
            pltpu.VMEM((chunks, _IDX_CHUNK), jnp.int32),
            pltpu.VMEM((g_per_w, _NUM_CLASSES), jnp.float32),
            pltpu.VMEM((g_per_w,), jnp.float32),
            pltpu.VMEM((rows_per_w, _NUM_CLASSES), jnp.float32),
            pltpu.SemaphoreType.DMA,
        ],
    )
    def kern(table_hbm, idx_hbm, w_hbm, out_hbm, idx_v, rows_v, w_v, out_v,
             sem):
        wid = lax.axis_index("s") * _NC + lax.axis_index("c")
        pltpu.sync_copy(idx_hbm.at[pl.ds(wid * chunks, chunks)], idx_v)
        pltpu.sync_copy(w_hbm.at[pl.ds(wid * g_per_w, g_per_w)], w_v)
        copies = []
        for c in range(chunks):
            copies.append(pltpu.async_copy(
                table_hbm.at[idx_v.at[c]],
                rows_v.at[pl.ds(c * _IDX_CHUNK, _IDX_CHUNK)],
                sem))
        for cp in copies:
            cp.wait()

        def body(p, _):
            wchunk = w_v[pl.ds(p * 16, 16)]
            for h in range(2):
                acc = jnp.zeros((_NUM_CLASSES,), jnp.float32)
                for k in range(_TOP_K):
                    ws = _splat_lane(wchunk, h * _TOP_K + k)
                    acc = acc + rows_v[p * 16 + h * _TOP_K + k] * ws
                out_v[2 * p + h] = acc
            return 0

        lax.fori_loop(0, rows_per_w // 2, body, 0)
        pltpu.sync_copy(out_v,
                        out_hbm.at[pl.ds(wid * rows_per_w, rows_per_w)])

    return kern(table, idx2d, wflat)


@jax.jit
def kernel(expert_probs, W1, b1, W2, b2, W3, b3):
    B = expert_probs.shape[0]
    flat8 = expert_probs.reshape(B, 8, 128)
    gidx, wts = _tc_gating(flat8, W1, b1, W2, b2, W3, b3)
    table = expert_probs.reshape(B * _NUM_EXPERTS, _NUM_CLASSES)
    idx2d = gidx.reshape(B * _TOP_K // _IDX_CHUNK, _IDX_CHUNK)
    return _sc_combine(table, idx2d, wts.reshape(B * _TOP_K))


# speedup vs baseline: 3.9980x; 3.9980x over previous
import functools

import jax
import jax.numpy as jnp
import numpy as np
from jax.experimental import pallas as pl
from jax.experimental.pallas import tpu as pltpu

_BATCH = 16384
_NUM_EXPERTS = 64
_NUM_CLASSES = 16
_TOP_K = 8
_IN_DIM = _NUM_EXPERTS * _NUM_CLASSES
_BLOCK = 2048

_EXPAND = np.zeros((_NUM_EXPERTS, _IN_DIM), dtype=np.float32)
_EXPAND[np.arange(_IN_DIM) // _NUM_CLASSES, np.arange(_IN_DIM)] = 1.0
_COLLAPSE = np.zeros((_IN_DIM, _NUM_CLASSES), dtype=np.float32)
_COLLAPSE[np.arange(_IN_DIM), np.arange(_IN_DIM) % _NUM_CLASSES] = 1.0


def _moe_block_kernel(x_ref, w1_ref, b1_ref, w2_ref, b2_ref, w3_ref, b3_ref,
                      er_ref, cl_ref, out_ref):
    x = x_ref[...].reshape(_BLOCK, _IN_DIM)
    h = jnp.maximum(
        jnp.dot(x.astype(jnp.bfloat16), w1_ref[...].astype(jnp.bfloat16),
                preferred_element_type=jnp.float32)
        + b1_ref[...], 0.0)
    h = jnp.maximum(
        jnp.dot(h.astype(jnp.bfloat16), w2_ref[...].astype(jnp.bfloat16),
                preferred_element_type=jnp.float32)
        + b2_ref[...], 0.0)
    logits = (jnp.dot(h.astype(jnp.bfloat16), w3_ref[...].astype(jnp.bfloat16),
                      preferred_element_type=jnp.float32)
              + b3_ref[...])

    ew = logits
    sel = jnp.zeros(logits.shape, dtype=jnp.bool_)
    m = None
    for _ in range(_TOP_K):
        mx = jnp.max(ew, axis=1, keepdims=True)
        if m is None:
            m = mx
        hit = ew == mx
        sel = jnp.logical_or(sel, hit)
        ew = jnp.where(hit, -jnp.inf, ew)

    w = jnp.where(sel, jnp.exp(logits - m), 0.0)
    w = w / jnp.sum(w, axis=1, keepdims=True)

    w_full = jnp.dot(w, er_ref[...], preferred_element_type=jnp.float32)
    out_ref[...] = jnp.dot(x * w_full, cl_ref[...],
                           preferred_element_type=jnp.float32)


@jax.jit
def kernel(expert_probs, W1, b1, W2, b2, W3, b3):
    B = expert_probs.shape[0]
    flat8 = expert_probs.reshape(B, 8, 128)
    grid = (B // _BLOCK,)
    full = lambda shape: pl.BlockSpec(shape, lambda i: (0,) * len(shape))
    return pl.pallas_call(
        _moe_block_kernel,
        grid=grid,
        in_specs=[
            pl.BlockSpec((_BLOCK, 8, 128), lambda i: (i, 0, 0)),
            full(W1.shape),
            full(b1.shape),
            full(W2.shape),
            full(b2.shape),
            full(W3.shape),
            full(b3.shape),
            full(_EXPAND.shape),
            full(_COLLAPSE.shape),
        ],
        out_specs=pl.BlockSpec((_BLOCK, _NUM_CLASSES), lambda i: (i, 0)),
        out_shape=jax.ShapeDtypeStruct((B, _NUM_CLASSES), jnp.float32),
    )(flat8, W1, b1, W2, b2, W3, b3, jnp.asarray(_EXPAND),
      jnp.asarray(_COLLAPSE))
